# baseline (speedup 1.0000x reference)
import functools

import jax
import jax.numpy as jnp
from jax import lax
from jax.experimental import pallas as pl
from jax.experimental.pallas import tpu as pltpu
from jax.experimental.pallas import tpu_sc as plsc

N = 10000
E = 320000
D_IN = 128
D_HID = 256
D_OUT = 128

NCORE = 2
NSUB = 16
N_PAD = 10240
ROWS_PER_TILE = N_PAD // NSUB
DEG_PAD = 10240
DEG_TILE = DEG_PAD // NSUB
DEG_K = 2000
AGG_K = 120
NBUF = 3
E_PAD = 322560
ROW_BLK = 1024


def _sc_mesh():
    return plsc.VectorSubcoreMesh(core_axis_name="c", subcore_axis_name="s")


def _deg_body(dst_hbm, degp_hbm, didx, ones_v, zbuf, dacc):
    c = lax.axis_index("c")
    s = lax.axis_index("s")

    def fill_z(i, _):
        zbuf[pl.ds(i * 16, 16)] = jnp.zeros((16,), jnp.float32)
        return 0

    lax.fori_loop(0, DEG_TILE // 16, fill_z, 0)

    def fill_o(i, _):
        ones_v[pl.ds(i * 16, 16)] = jnp.ones((16,), jnp.float32)
        return 0

    lax.fori_loop(0, DEG_K // 16, fill_o, 0)

    pltpu.sync_copy(zbuf, dacc.at[pl.ds(s * DEG_TILE, DEG_TILE)])
    plsc.subcore_barrier()

    edges_per_core = E // NCORE
    edges_per_tile = edges_per_core // NSUB
    base = c * edges_per_core + s * edges_per_tile

    def body(i, _):
        pltpu.sync_copy(dst_hbm.at[pl.ds(base + i * DEG_K, DEG_K)], didx)
        pltpu.sync_copy(ones_v, dacc.at[didx], add=True)
        return 0

    lax.fori_loop(0, edges_per_tile // DEG_K, body, 0)
    plsc.subcore_barrier()
    pltpu.sync_copy(dacc.at[pl.ds(s * DEG_TILE, DEG_TILE)], degp_hbm.at[c, s])


_deg_call = pl.kernel(
    _deg_body,
    out_type=jax.ShapeDtypeStruct((NCORE, NSUB, DEG_TILE), jnp.float32),
    mesh=_sc_mesh(),
    scratch_types=[
        pltpu.VMEM((DEG_K,), jnp.int32),
        pltpu.VMEM((DEG_K,), jnp.float32),
        pltpu.VMEM((DEG_TILE,), jnp.float32),
        pltpu.VMEM_SHARED((DEG_PAD,), jnp.float32),
    ],
)


def _edge_loop(g_ref, src_hbm, dst_hbm, acc,
               sx, dx, rx, sems, base, n_chunks):
    def prefetch(c, b):
        off = base + c * AGG_K
        pltpu.sync_copy(src_hbm.at[pl.ds(off, AGG_K)], sx[b])
        pltpu.sync_copy(dst_hbm.at[pl.ds(off, AGG_K)], dx[b])
        pltpu.async_copy(g_ref.at[sx[b]], rx[b], sems[b])

    for b in range(NBUF - 1):
        prefetch(b, b)

    def outer(g, _):
        for b in range(NBUF):
            c = g * NBUF + b

            @pl.when(c + NBUF - 1 < n_chunks)
            def _():
                prefetch(c + NBUF - 1, (b + NBUF - 1) % NBUF)

            pltpu.make_async_copy(g_ref.at[sx[b]], rx[b], sems[b]).wait()
            pltpu.sync_copy(rx[b], acc.at[dx[b]], add=True)
        return 0

    lax.fori_loop(0, n_chunks // NBUF, outer, 0)


def _agg_body(dh, g0, g1, src_hbm, dst_hbm, o0, o1,
              s0, s1, s2, d0, d1, d2, r0, r1, r2, acc, sem0, sem1, sem2):
    c = lax.axis_index("c")
    s = lax.axis_index("s")
    edges_per_tile = E_PAD // NSUB

    def run(g_ref, o_ref):
        pltpu.sync_copy(
            g_ref.at[pl.ds(s * ROWS_PER_TILE, ROWS_PER_TILE)],
            acc.at[pl.ds(s * ROWS_PER_TILE, ROWS_PER_TILE)],
        )
        plsc.subcore_barrier()
        _edge_loop(g_ref, src_hbm, dst_hbm, acc,
                   (s0, s1, s2), (d0, d1, d2), (r0, r1, r2),
                   (sem0, sem1, sem2),
                   s * edges_per_tile, edges_per_tile // AGG_K)
        plsc.subcore_barrier()
        pltpu.sync_copy(
            acc.at[pl.ds(s * ROWS_PER_TILE, ROWS_PER_TILE)],
            o_ref.at[pl.ds(s * ROWS_PER_TILE, ROWS_PER_TILE)],
        )

    @pl.when(c == 0)
    def _():
        run(g0, o0)

    @pl.when(c == 1)
    def _():
        run(g1, o1)


def _make_agg(dh):
    return pl.kernel(
        functools.partial(_agg_body, dh),
        out_type=(
            jax.ShapeDtypeStruct((N_PAD, dh), jnp.float32),
            jax.ShapeDtypeStruct((N_PAD, dh), jnp.float32),
        ),
        mesh=_sc_mesh(),
        scratch_types=(
            [pltpu.VMEM((AGG_K,), jnp.int32)] * (2 * NBUF)
            + [pltpu.VMEM((AGG_K, dh), jnp.float32)] * NBUF
            + [pltpu.VMEM_SHARED((N_PAD, dh), jnp.float32)]
            + [pltpu.SemaphoreType.DMA] * NBUF
        ),
    )


_agg_hid = _make_agg(D_HID // 2)


def _agg_full_body(g, src_hbm, dst_hbm, o0, o1,
                   s0, s1, s2, d0, d1, d2, r0, r1, r2, acc, sem0, sem1, sem2):
    c = lax.axis_index("c")
    s = lax.axis_index("s")

    pltpu.sync_copy(
        g.at[pl.ds(s * ROWS_PER_TILE, ROWS_PER_TILE)],
        acc.at[pl.ds(s * ROWS_PER_TILE, ROWS_PER_TILE)],
    )
    plsc.subcore_barrier()

    edges_per_core = E_PAD // NCORE
    edges_per_tile = edges_per_core // NSUB
    base = c * edges_per_core + s * edges_per_tile
    _edge_loop(g, src_hbm, dst_hbm, acc,
               (s0, s1, s2), (d0, d1, d2), (r0, r1, r2), (sem0, sem1, sem2),
               base, edges_per_tile // AGG_K)
    plsc.subcore_barrier()

    @pl.when(c == 0)
    def _():
        pltpu.sync_copy(
            acc.at[pl.ds(s * ROWS_PER_TILE, ROWS_PER_TILE)],
            o0.at[pl.ds(s * ROWS_PER_TILE, ROWS_PER_TILE)],
        )

    @pl.when(c == 1)
    def _():
        pltpu.sync_copy(
            acc.at[pl.ds(s * ROWS_PER_TILE, ROWS_PER_TILE)],
            o1.at[pl.ds(s * ROWS_PER_TILE, ROWS_PER_TILE)],
        )


_agg_out = pl.kernel(
    _agg_full_body,
    out_type=(
        jax.ShapeDtypeStruct((N_PAD, D_OUT), jnp.float32),
        jax.ShapeDtypeStruct((N_PAD, D_OUT), jnp.float32),
    ),
    mesh=_sc_mesh(),
    scratch_types=(
        [pltpu.VMEM((AGG_K,), jnp.int32)] * (2 * NBUF)
        + [pltpu.VMEM((AGG_K, D_OUT), jnp.float32)] * NBUF
        + [pltpu.VMEM_SHARED((N_PAD, D_OUT), jnp.float32)]
        + [pltpu.SemaphoreType.DMA] * NBUF
    ),
)


def _dinv_body(degp_ref, out_ref):
    out_ref[...] = lax.rsqrt(degp_ref[0] + degp_ref[1] + 1.0)


def _tc1_body(x_ref, w_ref, dinv_ref, g0_ref, g1_ref):
    h = jnp.dot(x_ref[...], w_ref[...], preferred_element_type=jnp.float32)
    g = h * dinv_ref[...]
    g0_ref[...] = g[:, : D_HID // 2]
    g1_ref[...] = g[:, D_HID // 2 :]


def _tc2_body(a0_ref, a1_ref, dinv_ref, b1_ref, w2_ref, g2_ref):
    agg = jnp.concatenate([a0_ref[...], a1_ref[...]], axis=1)
    h = agg * dinv_ref[...] + b1_ref[...]
    h = jnp.where(h > 0, h, jnp.exp(jnp.minimum(h, 0.0)) - 1.0)
    h2 = jnp.dot(h, w2_ref[...], preferred_element_type=jnp.float32)
    g2_ref[...] = h2 * dinv_ref[...]


def _tc3_body(o0_ref, o1_ref, g2_ref, dinv_ref, b2_ref, out_ref):
    agg = o0_ref[...] + o1_ref[...] - g2_ref[...]
    h = agg * dinv_ref[...] + b2_ref[...]
    out_ref[...] = jnp.where(h > 0, h, jnp.exp(jnp.minimum(h, 0.0)) - 1.0)


_GRID = N_PAD // ROW_BLK


def _row_spec(d):
    return pl.BlockSpec((ROW_BLK, d), lambda i: (i, 0))


def _full_spec(r, d):
    return pl.BlockSpec((r, d), lambda i: (0, 0))


_dinv_call = pl.pallas_call(
    _dinv_body,
    out_shape=jax.ShapeDtypeStruct((DEG_PAD // 128, 128), jnp.float32),
    in_specs=[pl.BlockSpec((NCORE, DEG_PAD // 128, 128), lambda: (0, 0, 0))],
    out_specs=pl.BlockSpec((DEG_PAD // 128, 128), lambda: (0, 0)),
)

_tc1_call = pl.pallas_call(
    _tc1_body,
    grid=(_GRID,),
    out_shape=(
        jax.ShapeDtypeStruct((N_PAD, D_HID // 2), jnp.float32),
        jax.ShapeDtypeStruct((N_PAD, D_HID // 2), jnp.float32),
    ),
    in_specs=[
        _row_spec(D_IN),
        _full_spec(D_IN, D_HID),
        _row_spec(1),
    ],
    out_specs=(_row_spec(D_HID // 2), _row_spec(D_HID // 2)),
)

_tc2_call = pl.pallas_call(
    _tc2_body,
    grid=(_GRID,),
    out_shape=jax.ShapeDtypeStruct((N_PAD, D_OUT), jnp.float32),
    in_specs=[
        _row_spec(D_HID // 2),
        _row_spec(D_HID // 2),
        _row_spec(1),
        _full_spec(1, D_HID),
        _full_spec(D_HID, D_OUT),
    ],
    out_specs=_row_spec(D_OUT),
)

_tc3_call = pl.pallas_call(
    _tc3_body,
    grid=(_GRID,),
    out_shape=jax.ShapeDtypeStruct((N_PAD, D_OUT), jnp.float32),
    in_specs=[
        _row_spec(D_OUT),
        _row_spec(D_OUT),
        _row_spec(D_OUT),
        _row_spec(1),
        _full_spec(1, D_OUT),
    ],
    out_specs=_row_spec(D_OUT),
)


def kernel(x, edge_index, W1, b1, W2, b2):
    pad_iota = jnp.arange(E_PAD - E, dtype=jnp.int32)
    src = jnp.concatenate([edge_index[0], pad_iota % N])
    dst = jnp.concatenate([edge_index[1], N + pad_iota % (N_PAD - N)])

    degp = _deg_call(edge_index[1])
    degp = degp.reshape(NCORE, DEG_PAD // 128, 128)
    dinv = _dinv_call(degp)
    dinv = dinv.reshape(N_PAD, 1)

    xp = jnp.pad(x, ((0, N_PAD - N), (0, 0)))
    g0, g1 = _tc1_call(xp, W1, dinv)
    a0, a1 = _agg_hid(g0, g1, src, dst)
    g2 = _tc2_call(a0, a1, dinv, b1.reshape(1, -1), W2)
    o0, o1 = _agg_out(g2, src, dst)
    out = _tc3_call(o0, o1, g2, dinv, b2.reshape(1, -1))
    return out[:N]


# speedup vs baseline: 1.1145x; 1.1145x over previous
import functools

import jax
import jax.numpy as jnp
from jax import lax
from jax.experimental import pallas as pl
from jax.experimental.pallas import tpu as pltpu
from jax.experimental.pallas import tpu_sc as plsc

N = 10000
E = 320000
D_IN = 128
D_HID = 256
D_OUT = 128

NCORE = 2
NSUB = 16
N_PAD = 10240
ROWS_PER_TILE = N_PAD // NSUB
DEG_PAD = 10240
DEG_TILE = DEG_PAD // NSUB
DEG_K = 2000
AGG_K = 184
NBUF = 2
E_PAD = 329728
ROW_BLK = 1024


def _sc_mesh():
    return plsc.VectorSubcoreMesh(core_axis_name="c", subcore_axis_name="s")


def _deg_body(dst_hbm, degp_hbm, didx, ones_v, zbuf, dacc):
    c = lax.axis_index("c")
    s = lax.axis_index("s")

    def fill_z(i, _):
        zbuf[pl.ds(i * 16, 16)] = jnp.zeros((16,), jnp.float32)
        return 0

    lax.fori_loop(0, DEG_TILE // 16, fill_z, 0)

    def fill_o(i, _):
        ones_v[pl.ds(i * 16, 16)] = jnp.ones((16,), jnp.float32)
        return 0

    lax.fori_loop(0, DEG_K // 16, fill_o, 0)

    pltpu.sync_copy(zbuf, dacc.at[pl.ds(s * DEG_TILE, DEG_TILE)])
    plsc.subcore_barrier()

    edges_per_core = E // NCORE
    edges_per_tile = edges_per_core // NSUB
    base = c * edges_per_core + s * edges_per_tile

    def body(i, _):
        pltpu.sync_copy(dst_hbm.at[pl.ds(base + i * DEG_K, DEG_K)], didx)
        pltpu.sync_copy(ones_v, dacc.at[didx], add=True)
        return 0

    lax.fori_loop(0, edges_per_tile // DEG_K, body, 0)
    plsc.subcore_barrier()
    pltpu.sync_copy(dacc.at[pl.ds(s * DEG_TILE, DEG_TILE)], degp_hbm.at[c, s])


_deg_call = pl.kernel(
    _deg_body,
    out_type=jax.ShapeDtypeStruct((NCORE, NSUB, DEG_TILE), jnp.float32),
    mesh=_sc_mesh(),
    scratch_types=[
        pltpu.VMEM((DEG_K,), jnp.int32),
        pltpu.VMEM((DEG_K,), jnp.float32),
        pltpu.VMEM((DEG_TILE,), jnp.float32),
        pltpu.VMEM_SHARED((DEG_PAD,), jnp.float32),
    ],
)


def _edge_loop(g_ref, src_hbm, dst_hbm, acc,
               sx, dx, rx, sems, base, n_chunks):
    def prefetch(c, b):
        off = base + c * AGG_K
        pltpu.sync_copy(src_hbm.at[pl.ds(off, AGG_K)], sx[b])
        pltpu.sync_copy(dst_hbm.at[pl.ds(off, AGG_K)], dx[b])
        pltpu.async_copy(g_ref.at[sx[b]], rx[b], sems[b])

    for b in range(NBUF - 1):
        prefetch(b, b)

    def outer(g, _):
        for b in range(NBUF):
            c = g * NBUF + b

            @pl.when(c + NBUF - 1 < n_chunks)
            def _():
                prefetch(c + NBUF - 1, (b + NBUF - 1) % NBUF)

            pltpu.make_async_copy(g_ref.at[sx[b]], rx[b], sems[b]).wait()
            pltpu.sync_copy(rx[b], acc.at[dx[b]], add=True)
        return 0

    lax.fori_loop(0, n_chunks // NBUF, outer, 0)


def _agg_body(dh, g0, g1, src_hbm, dst_hbm, o0, o1, *scr):
    sx = scr[0:NBUF]
    dx = scr[NBUF:2 * NBUF]
    rx = scr[2 * NBUF:3 * NBUF]
    acc = scr[3 * NBUF]
    sems = scr[3 * NBUF + 1:]
    c = lax.axis_index("c")
    s = lax.axis_index("s")
    edges_per_tile = E_PAD // NSUB

    def run(g_ref, o_ref):
        pltpu.sync_copy(
            g_ref.at[pl.ds(s * ROWS_PER_TILE, ROWS_PER_TILE)],
            acc.at[pl.ds(s * ROWS_PER_TILE, ROWS_PER_TILE)],
        )
        plsc.subcore_barrier()
        _edge_loop(g_ref, src_hbm, dst_hbm, acc,
                   sx, dx, rx, sems,
                   s * edges_per_tile, edges_per_tile // AGG_K)
        plsc.subcore_barrier()
        pltpu.sync_copy(
            acc.at[pl.ds(s * ROWS_PER_TILE, ROWS_PER_TILE)],
            o_ref.at[pl.ds(s * ROWS_PER_TILE, ROWS_PER_TILE)],
        )

    @pl.when(c == 0)
    def _():
        run(g0, o0)

    @pl.when(c == 1)
    def _():
        run(g1, o1)


def _make_agg(dh):
    return pl.kernel(
        functools.partial(_agg_body, dh),
        out_type=(
            jax.ShapeDtypeStruct((N_PAD, dh), jnp.float32),
            jax.ShapeDtypeStruct((N_PAD, dh), jnp.float32),
        ),
        mesh=_sc_mesh(),
        scratch_types=(
            [pltpu.VMEM((AGG_K,), jnp.int32)] * (2 * NBUF)
            + [pltpu.VMEM((AGG_K, dh), jnp.float32)] * NBUF
            + [pltpu.VMEM_SHARED((N_PAD, dh), jnp.float32)]
            + [pltpu.SemaphoreType.DMA] * NBUF
        ),
    )


_agg_hid = _make_agg(D_HID // 2)


def _agg_full_body(g, src_hbm, dst_hbm, o0, o1, *scr):
    sx = scr[0:NBUF]
    dx = scr[NBUF:2 * NBUF]
    rx = scr[2 * NBUF:3 * NBUF]
    acc = scr[3 * NBUF]
    sems = scr[3 * NBUF + 1:]
    c = lax.axis_index("c")
    s = lax.axis_index("s")

    pltpu.sync_copy(
        g.at[pl.ds(s * ROWS_PER_TILE, ROWS_PER_TILE)],
        acc.at[pl.ds(s * ROWS_PER_TILE, ROWS_PER_TILE)],
    )
    plsc.subcore_barrier()

    edges_per_core = E_PAD // NCORE
    edges_per_tile = edges_per_core // NSUB
    base = c * edges_per_core + s * edges_per_tile
    _edge_loop(g, src_hbm, dst_hbm, acc,
               sx, dx, rx, sems,
               base, edges_per_tile // AGG_K)
    plsc.subcore_barrier()

    @pl.when(c == 0)
    def _():
        pltpu.sync_copy(
            acc.at[pl.ds(s * ROWS_PER_TILE, ROWS_PER_TILE)],
            o0.at[pl.ds(s * ROWS_PER_TILE, ROWS_PER_TILE)],
        )

    @pl.when(c == 1)
    def _():
        pltpu.sync_copy(
            acc.at[pl.ds(s * ROWS_PER_TILE, ROWS_PER_TILE)],
            o1.at[pl.ds(s * ROWS_PER_TILE, ROWS_PER_TILE)],
        )


_agg_out = pl.kernel(
    _agg_full_body,
    out_type=(
        jax.ShapeDtypeStruct((N_PAD, D_OUT), jnp.float32),
        jax.ShapeDtypeStruct((N_PAD, D_OUT), jnp.float32),
    ),
    mesh=_sc_mesh(),
    scratch_types=(
        [pltpu.VMEM((AGG_K,), jnp.int32)] * (2 * NBUF)
        + [pltpu.VMEM((AGG_K, D_OUT), jnp.float32)] * NBUF
        + [pltpu.VMEM_SHARED((N_PAD, D_OUT), jnp.float32)]
        + [pltpu.SemaphoreType.DMA] * NBUF
    ),
)


def _dinv_body(degp_ref, out_ref):
    out_ref[...] = lax.rsqrt(degp_ref[0] + degp_ref[1] + 1.0)


def _tc1_body(x_ref, w_ref, dinv_ref, g0_ref, g1_ref):
    h = jnp.dot(x_ref[...], w_ref[...], preferred_element_type=jnp.float32)
    g = h * dinv_ref[...]
    g0_ref[...] = g[:, : D_HID // 2]
    g1_ref[...] = g[:, D_HID // 2 :]


def _tc2_body(a0_ref, a1_ref, dinv_ref, b1_ref, w2_ref, g2_ref):
    agg = jnp.concatenate([a0_ref[...], a1_ref[...]], axis=1)
    h = agg * dinv_ref[...] + b1_ref[...]
    h = jnp.where(h > 0, h, jnp.exp(jnp.minimum(h, 0.0)) - 1.0)
    h2 = jnp.dot(h, w2_ref[...], preferred_element_type=jnp.float32)
    g2_ref[...] = h2 * dinv_ref[...]


def _tc3_body(o0_ref, o1_ref, g2_ref, dinv_ref, b2_ref, out_ref):
    agg = o0_ref[...] + o1_ref[...] - g2_ref[...]
    h = agg * dinv_ref[...] + b2_ref[...]
    out_ref[...] = jnp.where(h > 0, h, jnp.exp(jnp.minimum(h, 0.0)) - 1.0)


_GRID = N_PAD // ROW_BLK


def _row_spec(d):
    return pl.BlockSpec((ROW_BLK, d), lambda i: (i, 0))


def _full_spec(r, d):
    return pl.BlockSpec((r, d), lambda i: (0, 0))


_dinv_call = pl.pallas_call(
    _dinv_body,
    out_shape=jax.ShapeDtypeStruct((DEG_PAD // 128, 128), jnp.float32),
    in_specs=[pl.BlockSpec((NCORE, DEG_PAD // 128, 128), lambda: (0, 0, 0))],
    out_specs=pl.BlockSpec((DEG_PAD // 128, 128), lambda: (0, 0)),
)

_tc1_call = pl.pallas_call(
    _tc1_body,
    grid=(_GRID,),
    out_shape=(
        jax.ShapeDtypeStruct((N_PAD, D_HID // 2), jnp.float32),
        jax.ShapeDtypeStruct((N_PAD, D_HID // 2), jnp.float32),
    ),
    in_specs=[
        _row_spec(D_IN),
        _full_spec(D_IN, D_HID),
        _row_spec(1),
    ],
    out_specs=(_row_spec(D_HID // 2), _row_spec(D_HID // 2)),
)

_tc2_call = pl.pallas_call(
    _tc2_body,
    grid=(_GRID,),
    out_shape=jax.ShapeDtypeStruct((N_PAD, D_OUT), jnp.float32),
    in_specs=[
        _row_spec(D_HID // 2),
        _row_spec(D_HID // 2),
        _row_spec(1),
        _full_spec(1, D_HID),
        _full_spec(D_HID, D_OUT),
    ],
    out_specs=_row_spec(D_OUT),
)

_tc3_call = pl.pallas_call(
    _tc3_body,
    grid=(_GRID,),
    out_shape=jax.ShapeDtypeStruct((N_PAD, D_OUT), jnp.float32),
    in_specs=[
        _row_spec(D_OUT),
        _row_spec(D_OUT),
        _row_spec(D_OUT),
        _row_spec(1),
        _full_spec(1, D_OUT),
    ],
    out_specs=_row_spec(D_OUT),
)


def kernel(x, edge_index, W1, b1, W2, b2):
    pad_iota = jnp.arange(E_PAD - E, dtype=jnp.int32)
    src = jnp.concatenate([edge_index[0], pad_iota % N])
    dst = jnp.concatenate([edge_index[1], N + pad_iota % (N_PAD - N)])

    degp = _deg_call(edge_index[1])
    degp = degp.reshape(NCORE, DEG_PAD // 128, 128)
    dinv = _dinv_call(degp)
    dinv = dinv.reshape(N_PAD, 1)

    xp = jnp.pad(x, ((0, N_PAD - N), (0, 0)))
    g0, g1 = _tc1_call(xp, W1, dinv)
    a0, a1 = _agg_hid(g0, g1, src, dst)
    g2 = _tc2_call(a0, a1, dinv, b1.reshape(1, -1), W2)
    o0, o1 = _agg_out(g2, src, dst)
    out = _tc3_call(o0, o1, g2, dinv, b2.reshape(1, -1))
    return out[:N]
